# baseline (speedup 1.0000x reference)
import functools

import jax
import jax.numpy as jnp
from jax import lax
from jax.experimental import pallas as pl
from jax.experimental.pallas import tpu as pltpu
from jax.experimental.pallas import tpu_sc as plsc

_D = 64
_L = 16
_NC, _NS = 2, 16
_NW = _NC * _NS
_G = 128
_K = 4


@functools.partial(jax.jit, static_argnums=(2,))
def _embed_gather(idx2, table, nrow):
    r_per_w = nrow // _NW
    g_per_w = r_per_w // _K
    mesh = plsc.VectorSubcoreMesh(core_axis_name="c", subcore_axis_name="s")

    @functools.partial(
        pl.kernel,
        out_type=jax.ShapeDtypeStruct((nrow, _G, _D), jnp.float32),
        mesh=mesh,
        scratch_types=[
            pltpu.VMEM((2, _K, _G), jnp.int32),
            pltpu.VMEM((2, _K, _G, _D), jnp.float32),
            pltpu.SemaphoreType.DMA,
        ],
        compiler_params=pltpu.CompilerParams(use_tc_tiling_on_sc=False),
    )
    def k(idx_hbm, table_hbm, out_hbm, idx_v, rows_v, sem):
        wid = lax.axis_index("s") * _NC + lax.axis_index("c")
        rbase = wid * r_per_w

        def step(g, carry):
            roff = rbase + g * _K
            pltpu.sync_copy(idx_hbm.at[pl.ds(roff, _K)], idx_v.at[0])
            copies = [
                pltpu.async_copy(table_hbm.at[idx_v.at[0, j]], rows_v.at[0, j], sem)
                for j in range(_K)
            ]
            for c in copies:
                c.wait()

            def scale(r, c2):
                for j in range(_D // _L):
                    v = rows_v[0, r // _G, r % _G, pl.ds(j * _L, _L)]
                    rows_v[0, r // _G, r % _G, pl.ds(j * _L, _L)] = v * 8.0
                return c2

            lax.fori_loop(0, _K * _G, scale, 0, unroll=2)
            pltpu.sync_copy(rows_v.at[0], out_hbm.at[pl.ds(roff, _K)])
            return carry

        lax.fori_loop(0, g_per_w, step, 0)

    return k(idx2, table)


def kernel(x, table):
    B = x.shape[0] * x.shape[1]
    assert B % (_NW * _K * _G) == 0
    out = _embed_gather(x.reshape(B // _G, _G), table, B // _G)
    return out.reshape(x.shape[0], x.shape[1], _D)
